# Initial kernel scaffold; baseline (speedup 1.0000x reference)
import jax
import jax.numpy as jnp
from jax.experimental import pallas as pl


def kernel(feat0, feat1, feat2, mask_feat, adj0, adj1, mask_adj0, mask_adj1, nei0, nei1, W_fc0, b_fc0, W_fc1, b_fc1, W_fc2, b_fc2, W_agg0, W_agg1, W_g1, b_g1, W_g2, b_g2, W_att, b_att, a_att, W_proj, b_proj, W_m1, b_m1, W_m2, b_m2):
    raise NotImplementedError("write your pallas kernel here")


# speedup vs baseline: 6.7462x; 6.7462x over previous
import jax
import jax.numpy as jnp
from jax.experimental import pallas as pl

N, NA, NS = 1024, 4096, 60
F0, F1, F2 = 1902, 334, 64
H, D = 256, 64
TAU = 0.5
F0P, F1P, NSP = 1920, 384, 64


def _elu(x):
    return jnp.where(x > 0, x, jnp.exp(x) - 1.0)


def _normalize(x):
    nrm = jnp.sqrt(jnp.sum(x * x, axis=1, keepdims=True))
    return x / jnp.clip(nrm, 1e-12)


def _encode_kernel(feat0_ref, mask_ref, feat1_ref, w0_ref, b0_ref, w1_ref,
                   b1_ref, htar_ref, hmask_ref, hnei0_ref):
    w0 = w0_ref[...]
    b0 = b0_ref[...]
    htar_ref[...] = _elu(
        jnp.dot(feat0_ref[...], w0, preferred_element_type=jnp.float32) + b0)
    hmask_ref[...] = _elu(
        jnp.dot(mask_ref[...], w0, preferred_element_type=jnp.float32) + b0)
    hnei0_ref[...] = _elu(
        jnp.dot(feat1_ref[...], w1_ref[...],
                preferred_element_type=jnp.float32) + b1_ref[...])


def _agg_kernel(nei0_ref, hnei0_ref, nei1_ref, feat2_ref, w2_ref, b2_ref,
                agg0_ref, agg1_ref):
    nei0 = nei0_ref[...]
    cnt0 = jnp.sum(nei0, axis=1, keepdims=True)
    cnt0 = jnp.where(cnt0 > 0, cnt0, 1.0)
    agg0_ref[...] = jnp.dot(nei0, hnei0_ref[...],
                            preferred_element_type=jnp.float32) / cnt0
    hnei1 = _elu(jnp.dot(feat2_ref[...], w2_ref[...],
                         preferred_element_type=jnp.float32) + b2_ref[...])
    nei1 = nei1_ref[...]
    cnt1 = jnp.sum(nei1, axis=1, keepdims=True)
    cnt1 = jnp.where(cnt1 > 0, cnt1, 1.0)
    agg1_ref[...] = jnp.dot(nei1, hnei1,
                            preferred_element_type=jnp.float32) / cnt1


def _fuse_kernel(htar_ref, hmask_ref, agg0_ref, agg1_ref, adj0_ref, adj1_ref,
                 madj0_ref, madj1_ref, wagg0_ref, wagg1_ref, wg1_ref, bg1_ref,
                 wg2_ref, bg2_ref, watt_ref, batt_ref, aatt_ref, wproj_ref,
                 bproj_ref, zc_ref, zf_ref):
    f32 = jnp.float32
    h_tar = htar_ref[...]
    a0w = jnp.dot(agg0_ref[...], wagg0_ref[...], preferred_element_type=f32)
    a1w = jnp.dot(agg1_ref[...], wagg1_ref[...], preferred_element_type=f32)
    h_view0 = _elu(h_tar + a0w)
    h_view1 = _elu(h_tar + a1w)
    h_mask = hmask_ref[...]
    h_mask0 = _elu(h_mask + a0w)
    h_mask1 = _elu(h_mask + a1w)
    adj0 = adj0_ref[...]
    adj1 = adj1_ref[...]
    adj_mean = 0.5 * (adj0 + adj1)
    wg1 = wg1_ref[...]
    bg1 = bg1_ref[...]
    wg2 = wg2_ref[...]
    bg2 = bg2_ref[...]

    def gcn(x, adj):
        h = jax.nn.relu(
            jnp.dot(adj, jnp.dot(x, wg1, preferred_element_type=f32),
                    preferred_element_type=f32) + bg1)
        return jnp.dot(adj, jnp.dot(h, wg2, preferred_element_type=f32),
                       preferred_element_type=f32) + bg2

    z_coarse = gcn(h_tar, adj_mean)
    hf0 = _normalize(gcn(h_view0, adj0))
    hf1 = _normalize(gcn(h_mask0, madj0_ref[...]))
    hf2 = _normalize(gcn(h_view1, adj1))
    hf3 = _normalize(gcn(h_mask1, madj1_ref[...]))

    watt = watt_ref[...]
    batt = batt_ref[...]
    aatt = aatt_ref[...]

    def score(h):
        t = jnp.tanh(jnp.dot(h, watt, preferred_element_type=f32) + batt)
        return jnp.sum(jnp.dot(t, aatt, preferred_element_type=f32)) / N

    s0, s1, s2, s3 = score(hf0), score(hf1), score(hf2), score(hf3)
    m = jnp.maximum(jnp.maximum(s0, s1), jnp.maximum(s2, s3))
    e0, e1 = jnp.exp(s0 - m), jnp.exp(s1 - m)
    e2, e3 = jnp.exp(s2 - m), jnp.exp(s3 - m)
    tot = e0 + e1 + e2 + e3
    z_fine = (e0 * hf0 + e1 * hf1 + e2 * hf2 + e3 * hf3) / tot

    wproj = wproj_ref[...]
    bproj = bproj_ref[...]
    zc_ref[...] = _normalize(
        jnp.tanh(jnp.dot(z_coarse, wproj, preferred_element_type=f32) + bproj))
    zf_ref[...] = _normalize(
        jnp.tanh(jnp.dot(z_fine, wproj, preferred_element_type=f32) + bproj))


def _loss_kernel(zf_ref, zc_ref, wm1_ref, bm1_ref, wm2_ref, bm2_ref, out_ref):
    f32 = jnp.float32
    zf = zf_ref[...]
    zc = zc_ref[...]
    s = jnp.dot(zf, zc.T, preferred_element_type=f32) * (1.0 / TAU)
    e = jnp.exp(s)
    a = jnp.dot(zf, wm1_ref[...], preferred_element_type=f32) + bm1_ref[...]
    b = jnp.dot(zc, wm1_ref[...], preferred_element_type=f32)
    bt = b.T
    wm2 = wm2_ref[...]
    acc = jnp.full((N, N), bm2_ref[0, 0], dtype=f32)
    for k in range(16):
        acc = acc + jnp.tanh(a[:, k:k + 1] + bt[k:k + 1, :]) * wm2[0, k]
    weight = jax.nn.sigmoid(acc)
    den = jnp.sum(e * weight, axis=1)
    diag = jnp.sum(zf * zc, axis=1) * (1.0 / TAU)
    out_ref[...] = jnp.reshape(jnp.sum(jnp.log(den) - diag) / N, (1, 1))


def kernel(feat0, feat1, feat2, mask_feat, adj0, adj1, mask_adj0, mask_adj1,
           nei0, nei1, W_fc0, b_fc0, W_fc1, b_fc1, W_fc2, b_fc2, W_agg0,
           W_agg1, W_g1, b_g1, W_g2, b_g2, W_att, b_att, a_att, W_proj,
           b_proj, W_m1, b_m1, W_m2, b_m2):
    f32 = jnp.float32
    sds = jax.ShapeDtypeStruct
    feat0p = jnp.pad(feat0, ((0, 0), (0, F0P - F0)))
    maskp = jnp.pad(mask_feat, ((0, 0), (0, F0P - F0)))
    w0p = jnp.pad(W_fc0, ((0, F0P - F0), (0, 0)))
    feat1p = jnp.pad(feat1, ((0, 0), (0, F1P - F1)))
    w1p = jnp.pad(W_fc1, ((0, F1P - F1), (0, 0)))
    feat2p = jnp.pad(feat2, ((0, NSP - NS), (0, 0)))
    nei1p = jnp.pad(nei1, ((0, 0), (0, NSP - NS)))

    h_tar, h_mask, h_nei0 = pl.pallas_call(
        _encode_kernel,
        out_shape=(sds((N, H), f32), sds((N, H), f32), sds((NA, H), f32)),
    )(feat0p, maskp, feat1p, w0p, b_fc0.reshape(1, H), w1p,
      b_fc1.reshape(1, H))

    agg0, agg1 = pl.pallas_call(
        _agg_kernel,
        out_shape=(sds((N, H), f32), sds((N, H), f32)),
    )(nei0, h_nei0, nei1p, feat2p, W_fc2, b_fc2.reshape(1, H))

    zc, zf = pl.pallas_call(
        _fuse_kernel,
        out_shape=(sds((N, D), f32), sds((N, D), f32)),
    )(h_tar, h_mask, agg0, agg1, adj0, adj1, mask_adj0, mask_adj1, W_agg0,
      W_agg1, W_g1, b_g1.reshape(1, D), W_g2, b_g2.reshape(1, D), W_att,
      b_att.reshape(1, D), a_att.reshape(D, 1), W_proj, b_proj.reshape(1, D))

    loss = pl.pallas_call(
        _loss_kernel,
        out_shape=sds((1, 1), f32),
    )(zf, zc, W_m1, b_m1.reshape(1, 16), W_m2.reshape(1, 16),
      b_m2.reshape(1, 1))
    return loss[0, 0]


# speedup vs baseline: 9.4457x; 1.4001x over previous
import jax
import jax.numpy as jnp
from jax.experimental import pallas as pl

N, NA, NS = 1024, 4096, 60
F0, F1, F2 = 1902, 334, 64
H, D = 256, 64
TAU = 0.5

EG = 4
AG = 4


def _elu(x):
    return jnp.where(x > 0, x, jnp.exp(x) - 1.0)


def _normalize(x):
    nrm = jnp.sqrt(jnp.sum(x * x, axis=1, keepdims=True))
    return x / jnp.clip(nrm, 1e-12)


def _encode_kernel(feat0_ref, mask_ref, feat1_ref, w0_ref, b0_ref, w1_ref,
                   b1_ref, htar_ref, hmask_ref, hnei0_ref):
    w0 = w0_ref[...]
    b0 = b0_ref[...]
    htar_ref[...] = _elu(
        jnp.dot(feat0_ref[...], w0, preferred_element_type=jnp.float32) + b0)
    hmask_ref[...] = _elu(
        jnp.dot(mask_ref[...], w0, preferred_element_type=jnp.float32) + b0)
    hnei0_ref[...] = _elu(
        jnp.dot(feat1_ref[...], w1_ref[...],
                preferred_element_type=jnp.float32) + b1_ref[...])


def _agg_kernel(nei0_ref, hnei0_ref, nei1_ref, feat2_ref, w2_ref, b2_ref,
                agg0_ref, agg1_ref):
    nei0 = nei0_ref[...]
    cnt0 = jnp.sum(nei0, axis=1, keepdims=True)
    cnt0 = jnp.where(cnt0 > 0, cnt0, 1.0)
    agg0_ref[...] = jnp.dot(nei0, hnei0_ref[...],
                            preferred_element_type=jnp.float32) / cnt0
    hnei1 = _elu(jnp.dot(feat2_ref[...], w2_ref[...],
                         preferred_element_type=jnp.float32) + b2_ref[...])
    nei1 = nei1_ref[...]
    cnt1 = jnp.sum(nei1, axis=1, keepdims=True)
    cnt1 = jnp.where(cnt1 > 0, cnt1, 1.0)
    agg1_ref[...] = jnp.dot(nei1, hnei1,
                            preferred_element_type=jnp.float32) / cnt1


def _fuse_kernel(htar_ref, hmask_ref, agg0_ref, agg1_ref, adj0_ref, adj1_ref,
                 madj0_ref, madj1_ref, wagg0_ref, wagg1_ref, wg1_ref, bg1_ref,
                 wg2_ref, bg2_ref, watt_ref, batt_ref, aatt_ref, wproj_ref,
                 bproj_ref, zc_ref, zf_ref):
    f32 = jnp.float32
    h_tar = htar_ref[...]
    a0w = jnp.dot(agg0_ref[...], wagg0_ref[...], preferred_element_type=f32)
    a1w = jnp.dot(agg1_ref[...], wagg1_ref[...], preferred_element_type=f32)
    h_view0 = _elu(h_tar + a0w)
    h_view1 = _elu(h_tar + a1w)
    h_mask = hmask_ref[...]
    h_mask0 = _elu(h_mask + a0w)
    h_mask1 = _elu(h_mask + a1w)
    adj0 = adj0_ref[...]
    adj1 = adj1_ref[...]
    adj_mean = 0.5 * (adj0 + adj1)
    wg1 = wg1_ref[...]
    bg1 = bg1_ref[...]
    wg2 = wg2_ref[...]
    bg2 = bg2_ref[...]

    def gcn(x, adj):
        h = jax.nn.relu(
            jnp.dot(adj, jnp.dot(x, wg1, preferred_element_type=f32),
                    preferred_element_type=f32) + bg1)
        return jnp.dot(adj, jnp.dot(h, wg2, preferred_element_type=f32),
                       preferred_element_type=f32) + bg2

    z_coarse = gcn(h_tar, adj_mean)
    hf0 = _normalize(gcn(h_view0, adj0))
    hf1 = _normalize(gcn(h_mask0, madj0_ref[...]))
    hf2 = _normalize(gcn(h_view1, adj1))
    hf3 = _normalize(gcn(h_mask1, madj1_ref[...]))

    watt = watt_ref[...]
    batt = batt_ref[...]
    aatt = aatt_ref[...]

    def score(h):
        t = jnp.tanh(jnp.dot(h, watt, preferred_element_type=f32) + batt)
        return jnp.sum(jnp.dot(t, aatt, preferred_element_type=f32)) / N

    s0, s1, s2, s3 = score(hf0), score(hf1), score(hf2), score(hf3)
    m = jnp.maximum(jnp.maximum(s0, s1), jnp.maximum(s2, s3))
    e0, e1 = jnp.exp(s0 - m), jnp.exp(s1 - m)
    e2, e3 = jnp.exp(s2 - m), jnp.exp(s3 - m)
    tot = e0 + e1 + e2 + e3
    z_fine = (e0 * hf0 + e1 * hf1 + e2 * hf2 + e3 * hf3) / tot

    wproj = wproj_ref[...]
    bproj = bproj_ref[...]
    zc_ref[...] = _normalize(
        jnp.tanh(jnp.dot(z_coarse, wproj, preferred_element_type=f32) + bproj))
    zf_ref[...] = _normalize(
        jnp.tanh(jnp.dot(z_fine, wproj, preferred_element_type=f32) + bproj))


def _loss_kernel(zf_ref, zc_ref, wm1_ref, bm1_ref, wm2_ref, bm2_ref, out_ref):
    f32 = jnp.float32
    zf = zf_ref[...]
    zc = zc_ref[...]
    s = jnp.dot(zf, zc.T, preferred_element_type=f32) * (1.0 / TAU)
    e = jnp.exp(s)
    a = jnp.dot(zf, wm1_ref[...], preferred_element_type=f32) + bm1_ref[...]
    b = jnp.dot(zc, wm1_ref[...], preferred_element_type=f32)
    bt = b.T
    wm2 = wm2_ref[...]
    acc = jnp.full((N, N), bm2_ref[0, 0], dtype=f32)
    for k in range(16):
        acc = acc + jnp.tanh(a[:, k:k + 1] + bt[k:k + 1, :]) * wm2[0, k]
    weight = jax.nn.sigmoid(acc)
    den = jnp.sum(e * weight, axis=1)
    diag = jnp.sum(zf * zc, axis=1) * (1.0 / TAU)
    out_ref[...] = jnp.reshape(jnp.sum(jnp.log(den) - diag) / N, (1, 1))


def kernel(feat0, feat1, feat2, mask_feat, adj0, adj1, mask_adj0, mask_adj1,
           nei0, nei1, W_fc0, b_fc0, W_fc1, b_fc1, W_fc2, b_fc2, W_agg0,
           W_agg1, W_g1, b_g1, W_g2, b_g2, W_att, b_att, a_att, W_proj,
           b_proj, W_m1, b_m1, W_m2, b_m2):
    f32 = jnp.float32
    sds = jax.ShapeDtypeStruct
    full = lambda shape: pl.BlockSpec(shape, lambda i: (0, 0))
    rows = lambda r, c: pl.BlockSpec((r, c), lambda i: (i, 0))

    h_tar, h_mask, h_nei0 = pl.pallas_call(
        _encode_kernel,
        grid=(EG,),
        in_specs=[
            rows(N // EG, F0),
            rows(N // EG, F0),
            rows(NA // EG, F1),
            full((F0, H)),
            full((1, H)),
            full((F1, H)),
            full((1, H)),
        ],
        out_specs=(rows(N // EG, H), rows(N // EG, H), rows(NA // EG, H)),
        out_shape=(sds((N, H), f32), sds((N, H), f32), sds((NA, H), f32)),
    )(feat0, mask_feat, feat1, W_fc0, b_fc0.reshape(1, H), W_fc1,
      b_fc1.reshape(1, H))

    agg0, agg1 = pl.pallas_call(
        _agg_kernel,
        grid=(AG,),
        in_specs=[
            rows(N // AG, NA),
            full((NA, H)),
            rows(N // AG, NS),
            full((NS, F2)),
            full((F2, H)),
            full((1, H)),
        ],
        out_specs=(rows(N // AG, H), rows(N // AG, H)),
        out_shape=(sds((N, H), f32), sds((N, H), f32)),
    )(nei0, h_nei0, nei1, feat2, W_fc2, b_fc2.reshape(1, H))

    zc, zf = pl.pallas_call(
        _fuse_kernel,
        out_shape=(sds((N, D), f32), sds((N, D), f32)),
    )(h_tar, h_mask, agg0, agg1, adj0, adj1, mask_adj0, mask_adj1, W_agg0,
      W_agg1, W_g1, b_g1.reshape(1, D), W_g2, b_g2.reshape(1, D), W_att,
      b_att.reshape(1, D), a_att.reshape(D, 1), W_proj, b_proj.reshape(1, D))

    loss = pl.pallas_call(
        _loss_kernel,
        out_shape=sds((1, 1), f32),
    )(zf, zc, W_m1, b_m1.reshape(1, 16), W_m2.reshape(1, 16),
      b_m2.reshape(1, 1))
    return loss[0, 0]
